# baseline (speedup 1.0000x reference)
import jax
import jax.numpy as jnp
from jax.experimental import pallas as pl
from jax.experimental.pallas import tpu as pltpu

CAP = 65536
X_DIM = 128
Y0, Y1 = 32, 32
ITER_ROWS = CAP // 128

N_STATE_CHUNKS = 4
N_CAM_CHUNKS = 16
STATE_CHUNK = CAP // N_STATE_CHUNKS
CAM_CHUNK = CAP // N_CAM_CHUNKS


def _push_body(pos_ref, cnt_ref, state_row, cam_row, sb_in, cb_in, it_in,
               sb_out, cb_out, it_out, sem_s, sem_c, sem_rows):
    pos = pos_ref[0]
    r = pos // 128
    c = pos - r * 128
    row_ids = jax.lax.broadcasted_iota(jnp.int32, (ITER_ROWS, 128), 0)
    col_ids = jax.lax.broadcasted_iota(jnp.int32, (ITER_ROWS, 128), 1)
    hit = (row_ids == r) & (col_ids == c)
    it_out[...] = jnp.where(hit, cnt_ref[0], it_in[...])

    for i in range(N_STATE_CHUNKS):
        pltpu.make_async_copy(
            sb_in.at[pl.ds(i * STATE_CHUNK, STATE_CHUNK)],
            sb_out.at[pl.ds(i * STATE_CHUNK, STATE_CHUNK)],
            sem_s.at[i],
        ).start()
    for i in range(N_CAM_CHUNKS):
        pltpu.make_async_copy(
            cb_in.at[pl.ds(i * CAM_CHUNK, CAM_CHUNK)],
            cb_out.at[pl.ds(i * CAM_CHUNK, CAM_CHUNK)],
            sem_c.at[i],
        ).start()
    for i in range(N_STATE_CHUNKS):
        pltpu.make_async_copy(
            sb_in.at[pl.ds(i * STATE_CHUNK, STATE_CHUNK)],
            sb_out.at[pl.ds(i * STATE_CHUNK, STATE_CHUNK)],
            sem_s.at[i],
        ).wait()
    for i in range(N_CAM_CHUNKS):
        pltpu.make_async_copy(
            cb_in.at[pl.ds(i * CAM_CHUNK, CAM_CHUNK)],
            cb_out.at[pl.ds(i * CAM_CHUNK, CAM_CHUNK)],
            sem_c.at[i],
        ).wait()

    row_s = pltpu.make_async_copy(state_row, sb_out.at[pl.ds(pos, 1)],
                                  sem_rows.at[0])
    row_c = pltpu.make_async_copy(cam_row, cb_out.at[pl.ds(pos, 1)],
                                  sem_rows.at[1])
    row_s.start()
    row_c.start()
    row_s.wait()
    row_c.wait()


def kernel(state_buffer, cam_data_buffer, iter_buffer, position, state,
           cam_data, count):
    pos2 = position.reshape(1)
    cnt2 = count.reshape(1)
    state_row = state.reshape(1, X_DIM)
    cam_row = cam_data.reshape(1, Y0, Y1)
    iter2d = iter_buffer.reshape(ITER_ROWS, 128)

    out_sb, out_cb, out_it = pl.pallas_call(
        _push_body,
        in_specs=[
            pl.BlockSpec(memory_space=pltpu.SMEM),
            pl.BlockSpec(memory_space=pltpu.SMEM),
            pl.BlockSpec(memory_space=pl.ANY),
            pl.BlockSpec(memory_space=pl.ANY),
            pl.BlockSpec(memory_space=pl.ANY),
            pl.BlockSpec(memory_space=pl.ANY),
            pl.BlockSpec(memory_space=pltpu.VMEM),
        ],
        out_specs=[
            pl.BlockSpec(memory_space=pl.ANY),
            pl.BlockSpec(memory_space=pl.ANY),
            pl.BlockSpec(memory_space=pltpu.VMEM),
        ],
        out_shape=[
            jax.ShapeDtypeStruct((CAP, X_DIM), jnp.float32),
            jax.ShapeDtypeStruct((CAP, Y0, Y1), jnp.float32),
            jax.ShapeDtypeStruct((ITER_ROWS, 128), jnp.int32),
        ],
        scratch_shapes=[
            pltpu.SemaphoreType.DMA((N_STATE_CHUNKS,)),
            pltpu.SemaphoreType.DMA((N_CAM_CHUNKS,)),
            pltpu.SemaphoreType.DMA((2,)),
        ],
    )(pos2, cnt2, state_row, cam_row, state_buffer, cam_data_buffer, iter2d)

    new_position = jnp.remainder(position + 1, CAP)
    full_buffer = (position + 1) == CAP
    return (out_sb, out_cb, out_it.reshape(CAP), new_position, full_buffer)


# speedup vs baseline: 53.9543x; 53.9543x over previous
import jax
import jax.numpy as jnp
from jax.experimental import pallas as pl
from jax.experimental.pallas import tpu as pltpu

CAP = 65536
X_DIM = 128
Y0, Y1 = 32, 32

Y_FLAT = Y0 * Y1
GRID = 32
ROWS = CAP // GRID


def _push_body(pos_ref, cnt_ref, state_v, cam_v, sb_in, cb_in, it_in,
               sb_out, cb_out, it_out):
    i = pl.program_id(0)
    base = i * ROWS
    pos = pos_ref[0]
    local = pos - base

    sb_out[...] = sb_in[...]
    cb_out[...] = cb_in[...]
    it_out[...] = it_in[...]

    @pl.when((pos >= base) & (pos < base + ROWS))
    def _overwrite():
        sb_out[pl.ds(local, 1), :] = state_v[...]
        cb_out[pl.ds(local, 1), :] = cam_v[...]
        col = jax.lax.broadcasted_iota(jnp.int32, (1, 1, ROWS), 2)
        it_out[...] = jnp.where(col == local, cnt_ref[0], it_in[...])


def kernel(state_buffer, cam_data_buffer, iter_buffer, position, state,
           cam_data, count):
    pos2 = position.reshape(1)
    cnt2 = count.reshape(1)
    state_row = state.reshape(1, X_DIM)
    cam_row = cam_data.reshape(1, Y_FLAT)
    cam2d = cam_data_buffer.reshape(CAP, Y_FLAT)
    iter3d = iter_buffer.reshape(GRID, 1, ROWS)

    out_sb, out_cb, out_it = pl.pallas_call(
        _push_body,
        grid=(GRID,),
        in_specs=[
            pl.BlockSpec(memory_space=pltpu.SMEM),
            pl.BlockSpec(memory_space=pltpu.SMEM),
            pl.BlockSpec((1, X_DIM), lambda i: (0, 0)),
            pl.BlockSpec((1, Y_FLAT), lambda i: (0, 0)),
            pl.BlockSpec((ROWS, X_DIM), lambda i: (i, 0)),
            pl.BlockSpec((ROWS, Y_FLAT), lambda i: (i, 0)),
            pl.BlockSpec((1, 1, ROWS), lambda i: (i, 0, 0)),
        ],
        out_specs=[
            pl.BlockSpec((ROWS, X_DIM), lambda i: (i, 0)),
            pl.BlockSpec((ROWS, Y_FLAT), lambda i: (i, 0)),
            pl.BlockSpec((1, 1, ROWS), lambda i: (i, 0, 0)),
        ],
        out_shape=[
            jax.ShapeDtypeStruct((CAP, X_DIM), jnp.float32),
            jax.ShapeDtypeStruct((CAP, Y_FLAT), jnp.float32),
            jax.ShapeDtypeStruct((GRID, 1, ROWS), jnp.int32),
        ],
        compiler_params=pltpu.CompilerParams(
            dimension_semantics=("arbitrary",),
        ),
    )(pos2, cnt2, state_row, cam_row, state_buffer, cam2d, iter3d)

    new_position = jnp.remainder(position + 1, CAP)
    full_buffer = (position + 1) == CAP
    return (out_sb, out_cb.reshape(CAP, Y0, Y1), out_it.reshape(CAP),
            new_position, full_buffer)
